# baseline (speedup 1.0000x reference)
import functools
import math

import jax
import jax.numpy as jnp
from jax import lax
from jax.experimental import pallas as pl
from jax.experimental.pallas import tpu as pltpu
from jax.experimental.pallas import tpu_sc as plsc

_VOCAB = 1000000
_DIM = 64
_B = 16384
_L = 200
_N = _B * _L
_NC = 2
_NS = 16
_NW = _NC * _NS
_BATCHES_W = _B // _NW
_CHUNK = _L
_NGRP = 13
_SCALE = math.sqrt(_DIM)

_mesh = plsc.VectorSubcoreMesh(core_axis_name="c", subcore_axis_name="s")

_splat_dn = lax.GatherDimensionNumbers(
    offset_dims=(), collapsed_slice_dims=(0,), start_index_map=(0,))


def _splat(vec, j):
    idx = jnp.full((16, 1), j, jnp.int32)
    return lax.gather(vec, idx, _splat_dn, slice_sizes=(1,),
                      mode=lax.GatherScatterMode.PROMISE_IN_BOUNDS)


@functools.partial(
    pl.kernel,
    out_type=jax.ShapeDtypeStruct((_B, _L, _DIM), jnp.float32),
    mesh=_mesh,
    scratch_types=[
        pltpu.VMEM((_CHUNK,), jnp.int32),
        pltpu.VMEM((_CHUNK,), jnp.int32),
        pltpu.VMEM((_CHUNK,), jnp.int32),
        pltpu.VMEM((_CHUNK,), jnp.int32),
        pltpu.VMEM((_CHUNK, 2 * _DIM), jnp.float32),
        pltpu.VMEM((_CHUNK, 2 * _DIM), jnp.float32),
        pltpu.VMEM((_CHUNK, _DIM), jnp.float32),
        pltpu.VMEM((_CHUNK, _DIM), jnp.float32),
        pltpu.SemaphoreType.DMA,
        pltpu.SemaphoreType.DMA,
        pltpu.SemaphoreType.DMA,
        pltpu.SemaphoreType.DMA,
    ],
    compiler_params=pltpu.CompilerParams(
        needs_layout_passes=False, use_tc_tiling_on_sc=True),
)
def _embed_gather(table_hbm, idx_hbm, out_hbm,
                  idx0, idx1, hid0, hid1, rows0, rows1, ob0, ob1,
                  g0, g1, s0, s1):
    wid = lax.axis_index("s") * _NC + lax.axis_index("c")
    base_b = wid * _BATCHES_W
    base_i = base_b * _L
    idx_v = (idx0, idx1)
    hid_v = (hid0, hid1)
    rows_v = (rows0, rows1)
    out_v = (ob0, ob1)
    gsem = (g0, g1)
    ssem = (s0, s1)
    lane = lax.iota(jnp.int32, 16)

    def stage(i, b):
        pltpu.sync_copy(idx_hbm.at[pl.ds(base_i + i * _CHUNK, _CHUNK)], idx_v[b])

        def halve(g, carry):
            r0 = lax.min(g * 16, _CHUNK - 16)
            sl = pl.ds(r0, 16)
            hid_v[b][sl] = lax.shift_right_logical(idx_v[b][sl], 1)
            return carry

        lax.fori_loop(0, _NGRP, halve, 0)
        pltpu.async_copy(table_hbm.at[hid_v[b]], rows_v[b], gsem[b])

    def select_scale(b):
        rows = rows_v[b]
        idxv = idx_v[b]
        outb = out_v[b]

        def grp(g, carry):
            r0 = lax.min(g * 16, _CHUNK - 16)
            idx16 = idxv[pl.ds(r0, 16)]
            par64 = (idx16 & 1) << 6
            for j in range(16):
                r = r0 + j
                cols = _splat(par64, j) + lane
                row16 = jnp.full((16,), 1, jnp.int32) * r
                for cc in range(_DIM // 16):
                    val = plsc.load_gather(rows, [row16, cols + cc * 16]) * _SCALE
                    outb[r, pl.ds(cc * 16, 16)] = val
            return carry

        lax.fori_loop(0, _NGRP, grp, 0)

    stage(0, 0)

    def outer(g, carry):
        for b in range(2):
            i = 2 * g + b
            nb = 1 - b
            pltpu.make_async_copy(table_hbm.at[hid_v[b]], rows_v[b], gsem[b]).wait()
            if b == 0:
                @pl.when(g > 0)
                def _wait_prev_store():
                    pltpu.make_async_copy(
                        out_v[nb], out_hbm.at[base_b], ssem[nb]).wait()
                stage(i + 1, nb)
            else:
                @pl.when(g < _BATCHES_W // 2 - 1)
                def _prefetch():
                    pltpu.make_async_copy(
                        out_v[nb], out_hbm.at[base_b], ssem[nb]).wait()
                    stage(i + 1, nb)
            select_scale(b)
            pltpu.async_copy(out_v[b], out_hbm.at[base_b + i], ssem[b])
        return carry

    lax.fori_loop(0, _BATCHES_W // 2, outer, 0)
    pltpu.make_async_copy(ob0, out_hbm.at[base_b], s0).wait()
    pltpu.make_async_copy(ob1, out_hbm.at[base_b], s1).wait()


def kernel(tokens, table):
    flat = tokens.reshape(_N)
    table2 = table.reshape(_VOCAB // 2, 2 * _DIM)
    return _embed_gather(table2, flat)


# speedup vs baseline: 1.4540x; 1.4540x over previous
import functools
import math

import jax
import jax.numpy as jnp
from jax import lax
from jax.experimental import pallas as pl
from jax.experimental.pallas import tpu as pltpu
from jax.experimental.pallas import tpu_sc as plsc

_VOCAB = 1000000
_DIM = 64
_B = 16384
_L = 200
_N = _B * _L
_NC = 2
_NS = 16
_NW = _NC * _NS
_PER_W = _N // _NW
_CHUNK = 800
_STEPS = _PER_W // _CHUNK
_SCALE = math.sqrt(_DIM)

_mesh = plsc.VectorSubcoreMesh(core_axis_name="c", subcore_axis_name="s")


@functools.partial(
    pl.kernel,
    out_type=jax.ShapeDtypeStruct((_N, _DIM), jnp.float32),
    mesh=_mesh,
    scratch_types=[
        pltpu.VMEM((_CHUNK,), jnp.int32),
        pltpu.VMEM((_CHUNK,), jnp.int32),
        pltpu.VMEM((_CHUNK, _DIM), jnp.float32),
        pltpu.VMEM((_CHUNK, _DIM), jnp.float32),
        pltpu.SemaphoreType.DMA,
        pltpu.SemaphoreType.DMA,
        pltpu.SemaphoreType.DMA,
        pltpu.SemaphoreType.DMA,
    ],
    compiler_params=pltpu.CompilerParams(
        needs_layout_passes=False, use_tc_tiling_on_sc=False),
)
def _embed_gather(table_hbm, idx_hbm, out_hbm,
                  idx0, idx1, rows0, rows1, g0, g1, s0, s1):
    wid = lax.axis_index("s") * _NC + lax.axis_index("c")
    base = wid * _PER_W
    idx_v = (idx0, idx1)
    rows_v = (rows0, rows1)
    gsem = (g0, g1)
    ssem = (s0, s1)

    def chunk_off(i):
        return base + i * _CHUNK

    def stage(i, b):
        pltpu.sync_copy(idx_hbm.at[pl.ds(chunk_off(i), _CHUNK)], idx_v[b])
        pltpu.async_copy(table_hbm.at[idx_v[b]], rows_v[b], gsem[b])

    def scale_rows(rv):
        def scale_row(r, carry):
            for c in range(_DIM // 16):
                sl = pl.ds(c * 16, 16)
                rv[r, sl] = rv[r, sl] * _SCALE
            return carry
        lax.fori_loop(0, _CHUNK, scale_row, 0, unroll=4)

    stage(0, 0)

    def outer(g, carry):
        for b in range(2):
            i = 2 * g + b
            nb = 1 - b
            pltpu.make_async_copy(table_hbm.at[idx_v[b]], rows_v[b], gsem[b]).wait()
            if b == 0:
                @pl.when(g > 0)
                def _wait_prev_store():
                    pltpu.make_async_copy(
                        rows_v[nb], out_hbm.at[pl.ds(chunk_off(0), _CHUNK)], ssem[nb]
                    ).wait()
                stage(i + 1, nb)
            else:
                @pl.when(g < _STEPS // 2 - 1)
                def _prefetch():
                    pltpu.make_async_copy(
                        rows_v[nb], out_hbm.at[pl.ds(chunk_off(0), _CHUNK)], ssem[nb]
                    ).wait()
                    stage(i + 1, nb)
            scale_rows(rows_v[b])
            pltpu.async_copy(
                rows_v[b], out_hbm.at[pl.ds(chunk_off(i), _CHUNK)], ssem[b])
        return carry

    lax.fori_loop(0, _STEPS // 2, outer, 0)
    pltpu.make_async_copy(rows0, out_hbm.at[pl.ds(base, _CHUNK)], s0).wait()
    pltpu.make_async_copy(rows1, out_hbm.at[pl.ds(base, _CHUNK)], s1).wait()


def kernel(tokens, table):
    flat = tokens.reshape(_N)
    out = _embed_gather(table, flat)
    return out.reshape(_B, _L, _DIM)
